# Initial kernel scaffold; baseline (speedup 1.0000x reference)
import jax
import jax.numpy as jnp
from jax.experimental import pallas as pl


def kernel(classifications, regressions, anchors, annotations):
    raise NotImplementedError("write your pallas kernel here")


# speedup vs baseline: 2.9849x; 2.9849x over previous
import functools

import jax
import jax.numpy as jnp
from jax.experimental import pallas as pl


def _body(cls_ref, reg_ref, anc_ref, ann_ref, cls_out, reg_out, np_out):
    i = pl.program_id(1)

    ann = ann_ref[0]
    gx1 = ann[0:1, :]
    gy1 = ann[1:2, :]
    gx2 = ann[2:3, :]
    gy2 = ann[3:4, :]
    glab = ann[4:5, :]

    anc = anc_ref[...]
    ax1 = anc[:, 0:1]
    ay1 = anc[:, 1:2]
    ax2 = anc[:, 2:3]
    ay2 = anc[:, 3:4]

    iw = jnp.maximum(jnp.minimum(ax2, gx2) - jnp.maximum(ax1, gx1), 0.0)
    ih = jnp.maximum(jnp.minimum(ay2, gy2) - jnp.maximum(ay1, gy1), 0.0)
    ia = iw * ih
    aarea = (ax2 - ax1) * (ay2 - ay1)
    garea = (gx2 - gx1) * (gy2 - gy1)
    iou = ia / (aarea + garea - ia)

    imax = jnp.max(iou, axis=1, keepdims=True)
    blk, m = iou.shape
    jidx = jax.lax.broadcasted_iota(jnp.int32, (blk, m), 1)
    iarg = jnp.min(jnp.where(iou == imax, jidx, m), axis=1, keepdims=True)
    sel = (jidx == iarg).astype(jnp.float32)

    bx1 = jnp.sum(sel * gx1, axis=1, keepdims=True)
    by1 = jnp.sum(sel * gy1, axis=1, keepdims=True)
    bx2 = jnp.sum(sel * gx2, axis=1, keepdims=True)
    by2 = jnp.sum(sel * gy2, axis=1, keepdims=True)
    lab = jnp.sum(sel * glab, axis=1, keepdims=True)

    posf = (imax > 0.5).astype(jnp.float32)
    incf = jnp.maximum(posf, (imax < 0.4).astype(jnp.float32))

    p = jnp.clip(cls_ref[0], 1e-4, 1.0 - 1e-4)
    fl0 = (-0.25) * p * p * jnp.log(1.0 - p)
    row0 = jnp.sum(fl0, axis=1, keepdims=True)
    c = p.shape[1]
    lane = jax.lax.broadcasted_iota(jnp.int32, (blk, c), 1)
    eql = (lane == lab.astype(jnp.int32)).astype(jnp.float32)
    plab = jnp.sum(eql * p, axis=1, keepdims=True)
    plab = jnp.clip(plab, 1e-4, 1.0 - 1e-4)
    fl1 = (-0.25) * (1.0 - plab) * (1.0 - plab) * jnp.log(plab)
    fl0l = (-0.25) * plab * plab * jnp.log(1.0 - plab)
    cls_part = jnp.sum(incf * row0 + posf * (fl1 - fl0l), axis=0,
                       keepdims=True)

    aw = ax2 - ax1
    ah = ay2 - ay1
    acx = ax1 + 0.5 * aw
    acy = ay1 + 0.5 * ah
    gw = jnp.clip(bx2 - bx1, 1.0, None)
    gh = jnp.clip(by2 - by1, 1.0, None)
    gcx = bx1 + 0.5 * gw
    gcy = by1 + 0.5 * gh
    dx = (gcx - acx) / aw / 0.1
    dy = (gcy - acy) / ah / 0.1
    dw = jnp.log(gw / aw) / 0.2
    dh = jnp.log(gh / ah) / 0.2
    rt = jnp.concatenate([dx, dy, dw, dh], axis=1)
    d = reg_ref[0] - rt
    ad = jnp.abs(d)
    sm = jnp.where(ad < 1.0, 0.5 * d * d, ad - 0.5)
    reg_part = jnp.sum(jnp.sum(sm * posf, axis=1, keepdims=True), axis=0,
                       keepdims=True)
    np_part = jnp.sum(posf, axis=0, keepdims=True)

    @pl.when(i == 0)
    def _init():
        cls_out[0] = cls_part
        reg_out[0] = reg_part
        np_out[0] = np_part

    @pl.when(i != 0)
    def _acc():
        cls_out[0] += cls_part
        reg_out[0] += reg_part
        np_out[0] += np_part


def _final(cs_ref, rs_ref, np_ref, co_ref, ro_ref):
    npv = np_ref[...]
    b = npv.shape[0]
    npc = jnp.maximum(npv, 1.0)
    cl = cs_ref[...] / npc
    rl = jnp.where(npv > 0.0, rs_ref[...] / (npc * 4.0), 0.0)
    co_ref[...] = jnp.sum(cl, axis=0, keepdims=True) / float(b)
    ro_ref[...] = jnp.sum(rl, axis=0, keepdims=True) / float(b)


@jax.jit
def kernel(classifications, regressions, anchors, annotations):
    b, n, c = classifications.shape
    m = annotations.shape[1]
    blk = 2000
    nblk = n // blk

    ann_t = jnp.transpose(annotations, (0, 2, 1))
    anchor = anchors[0]

    f32 = jnp.float32
    cs, rs, npos = pl.pallas_call(
        _body,
        grid=(b, nblk),
        in_specs=[
            pl.BlockSpec((1, blk, c), lambda bi, ii: (bi, ii, 0)),
            pl.BlockSpec((1, blk, 4), lambda bi, ii: (bi, ii, 0)),
            pl.BlockSpec((blk, 4), lambda bi, ii: (ii, 0)),
            pl.BlockSpec((1, 5, m), lambda bi, ii: (bi, 0, 0)),
        ],
        out_specs=[
            pl.BlockSpec((1, 1, 1), lambda bi, ii: (bi, 0, 0)),
            pl.BlockSpec((1, 1, 1), lambda bi, ii: (bi, 0, 0)),
            pl.BlockSpec((1, 1, 1), lambda bi, ii: (bi, 0, 0)),
        ],
        out_shape=[
            jax.ShapeDtypeStruct((b, 1, 1), f32),
            jax.ShapeDtypeStruct((b, 1, 1), f32),
            jax.ShapeDtypeStruct((b, 1, 1), f32),
        ],
    )(classifications, regressions, anchor, ann_t)
    cs = cs.reshape(b, 1)
    rs = rs.reshape(b, 1)
    npos = npos.reshape(b, 1)

    co, ro = pl.pallas_call(
        _final,
        out_shape=[
            jax.ShapeDtypeStruct((1, 1), f32),
            jax.ShapeDtypeStruct((1, 1), f32),
        ],
    )(cs, rs, npos)
    return co.reshape(1), ro.reshape(1)


# speedup vs baseline: 3.3760x; 1.1310x over previous
import functools

import jax
import jax.numpy as jnp
from jax.experimental import pallas as pl
from jax.experimental.pallas import tpu as pltpu


def _body(cls_ref, reg_ref, anc_ref, ann_ref, annm_ref, cls_out, reg_out,
          np_out):
    i = pl.program_id(1)

    ann = ann_ref[0]
    gx1 = ann[0:1, :]
    gy1 = ann[1:2, :]
    gx2 = ann[2:3, :]
    gy2 = ann[3:4, :]
    glab = ann[4:5, :]

    anc = anc_ref[...]
    ax1 = anc[:, 0:1]
    ay1 = anc[:, 1:2]
    ax2 = anc[:, 2:3]
    ay2 = anc[:, 3:4]

    iw = jnp.maximum(jnp.minimum(ax2, gx2) - jnp.maximum(ax1, gx1), 0.0)
    ih = jnp.maximum(jnp.minimum(ay2, gy2) - jnp.maximum(ay1, gy1), 0.0)
    ia = iw * ih
    aarea = (ax2 - ax1) * (ay2 - ay1)
    garea = (gx2 - gx1) * (gy2 - gy1)
    iou = ia / (aarea + garea - ia)

    imax = jnp.max(iou, axis=1, keepdims=True)
    blk, m = iou.shape
    jidx = jax.lax.broadcasted_iota(jnp.int32, (blk, m), 1)
    iarg = jnp.min(jnp.where(iou == imax, jidx, m), axis=1, keepdims=True)
    sel = (jidx == iarg).astype(jnp.float32)

    assigned = jnp.dot(sel, annm_ref[0],
                       preferred_element_type=jnp.float32)
    bx1 = assigned[:, 0:1]
    by1 = assigned[:, 1:2]
    bx2 = assigned[:, 2:3]
    by2 = assigned[:, 3:4]
    lab = assigned[:, 4:5]

    posf = (imax > 0.5).astype(jnp.float32)
    incf = jnp.maximum(posf, (imax < 0.4).astype(jnp.float32))

    p = jnp.clip(cls_ref[0], 1e-4, 1.0 - 1e-4)
    fl0 = (-0.25) * p * p * jnp.log(1.0 - p)
    row0 = jnp.sum(fl0, axis=1, keepdims=True)
    c = p.shape[1]
    lane = jax.lax.broadcasted_iota(jnp.int32, (blk, c), 1)
    eql = (lane == lab.astype(jnp.int32)).astype(jnp.float32)
    plab = jnp.sum(eql * p, axis=1, keepdims=True)
    plab = jnp.clip(plab, 1e-4, 1.0 - 1e-4)
    fl1 = (-0.25) * (1.0 - plab) * (1.0 - plab) * jnp.log(plab)
    fl0l = (-0.25) * plab * plab * jnp.log(1.0 - plab)
    cls_part = jnp.sum(incf * row0 + posf * (fl1 - fl0l), axis=0,
                       keepdims=True)

    aw = ax2 - ax1
    ah = ay2 - ay1
    acx = ax1 + 0.5 * aw
    acy = ay1 + 0.5 * ah
    gw = jnp.clip(bx2 - bx1, 1.0, None)
    gh = jnp.clip(by2 - by1, 1.0, None)
    gcx = bx1 + 0.5 * gw
    gcy = by1 + 0.5 * gh
    dx = (gcx - acx) / aw / 0.1
    dy = (gcy - acy) / ah / 0.1
    dw = jnp.log(gw / aw) / 0.2
    dh = jnp.log(gh / ah) / 0.2
    rt = jnp.concatenate([dx, dy, dw, dh], axis=1)
    d = reg_ref[0] - rt
    ad = jnp.abs(d)
    sm = jnp.where(ad < 1.0, 0.5 * d * d, ad - 0.5)
    reg_part = jnp.sum(jnp.sum(sm * posf, axis=1, keepdims=True), axis=0,
                       keepdims=True)
    np_part = jnp.sum(posf, axis=0, keepdims=True)

    @pl.when(i == 0)
    def _init():
        cls_out[0] = cls_part
        reg_out[0] = reg_part
        np_out[0] = np_part

    @pl.when(i != 0)
    def _acc():
        cls_out[0] += cls_part
        reg_out[0] += reg_part
        np_out[0] += np_part


def _final(cs_ref, rs_ref, np_ref, co_ref, ro_ref):
    npv = np_ref[...]
    b = npv.shape[0]
    npc = jnp.maximum(npv, 1.0)
    cl = cs_ref[...] / npc
    rl = jnp.where(npv > 0.0, rs_ref[...] / (npc * 4.0), 0.0)
    co_ref[...] = jnp.sum(cl, axis=0, keepdims=True) / float(b)
    ro_ref[...] = jnp.sum(rl, axis=0, keepdims=True) / float(b)


@jax.jit
def kernel(classifications, regressions, anchors, annotations):
    b, n, c = classifications.shape
    m = annotations.shape[1]
    blk = 2000
    nblk = n // blk

    ann_t = jnp.transpose(annotations, (0, 2, 1))
    anchor = anchors[0]

    f32 = jnp.float32
    cs, rs, npos = pl.pallas_call(
        _body,
        grid=(b, nblk),
        in_specs=[
            pl.BlockSpec((1, blk, c), lambda bi, ii: (bi, ii, 0)),
            pl.BlockSpec((1, blk, 4), lambda bi, ii: (bi, ii, 0)),
            pl.BlockSpec((blk, 4), lambda bi, ii: (ii, 0)),
            pl.BlockSpec((1, 5, m), lambda bi, ii: (bi, 0, 0)),
            pl.BlockSpec((1, m, 5), lambda bi, ii: (bi, 0, 0)),
        ],
        out_specs=[
            pl.BlockSpec((1, 1, 1), lambda bi, ii: (bi, 0, 0)),
            pl.BlockSpec((1, 1, 1), lambda bi, ii: (bi, 0, 0)),
            pl.BlockSpec((1, 1, 1), lambda bi, ii: (bi, 0, 0)),
        ],
        out_shape=[
            jax.ShapeDtypeStruct((b, 1, 1), f32),
            jax.ShapeDtypeStruct((b, 1, 1), f32),
            jax.ShapeDtypeStruct((b, 1, 1), f32),
        ],
        compiler_params=pltpu.CompilerParams(
            dimension_semantics=("parallel", "arbitrary")),
    )(classifications, regressions, anchor, ann_t, annotations)
    cs = cs.reshape(b, 1)
    rs = rs.reshape(b, 1)
    npos = npos.reshape(b, 1)

    co, ro = pl.pallas_call(
        _final,
        out_shape=[
            jax.ShapeDtypeStruct((1, 1), f32),
            jax.ShapeDtypeStruct((1, 1), f32),
        ],
    )(cs, rs, npos)
    return co.reshape(1), ro.reshape(1)


# speedup vs baseline: 5.8739x; 1.7399x over previous
import jax
import jax.numpy as jnp
from jax.experimental import pallas as pl
from jax.experimental.pallas import tpu as pltpu


def _body(cls_ref, regt_ref, anct_ref, ann5_ref, annm_ref, cls_out,
          reg_out, np_out):
    i = pl.program_id(1)
    f32 = jnp.float32

    annm = annm_ref[0]
    gx1 = annm[:, 0:1]
    gy1 = annm[:, 1:2]
    gx2 = annm[:, 2:3]
    gy2 = annm[:, 3:4]

    anct = anct_ref[0]
    ax1 = anct[0:1, :]
    ay1 = anct[1:2, :]
    ax2 = anct[2:3, :]
    ay2 = anct[3:4, :]

    iw = jnp.maximum(jnp.minimum(ax2, gx2) - jnp.maximum(ax1, gx1), 0.0)
    ih = jnp.maximum(jnp.minimum(ay2, gy2) - jnp.maximum(ay1, gy1), 0.0)
    ia = iw * ih
    aarea = (ax2 - ax1) * (ay2 - ay1)
    garea = (gx2 - gx1) * (gy2 - gy1)
    iou = ia / (aarea + garea - ia)

    m, blk = iou.shape
    imax = jnp.max(iou, axis=0, keepdims=True)
    jidx = jax.lax.broadcasted_iota(jnp.int32, (m, blk), 0)
    iarg = jnp.min(jnp.where(iou == imax, jidx, m), axis=0, keepdims=True)
    sel = (jidx == iarg).astype(f32)

    assigned = jnp.dot(ann5_ref[0], sel,
                       precision=jax.lax.Precision.HIGHEST,
                       preferred_element_type=f32)
    bx1 = assigned[0:1, :]
    by1 = assigned[1:2, :]
    bx2 = assigned[2:3, :]
    by2 = assigned[3:4, :]
    lab = assigned[4:5, :]

    posf = (imax > 0.5).astype(f32)
    incf = jnp.maximum(posf, (imax < 0.4).astype(f32))

    aw = ax2 - ax1
    ah = ay2 - ay1
    acx = ax1 + 0.5 * aw
    acy = ay1 + 0.5 * ah
    gw = jnp.clip(bx2 - bx1, 1.0, None)
    gh = jnp.clip(by2 - by1, 1.0, None)
    gcx = bx1 + 0.5 * gw
    gcy = by1 + 0.5 * gh
    dx = (gcx - acx) / aw / 0.1
    dy = (gcy - acy) / ah / 0.1
    dw = jnp.log(gw / aw) / 0.2
    dh = jnp.log(gh / ah) / 0.2
    rt = jnp.concatenate([dx, dy, dw, dh], axis=0)
    d = regt_ref[0, 0] - rt
    ad = jnp.abs(d)
    sm = jnp.where(ad < 1.0, 0.5 * d * d, ad - 0.5)
    smrow = jnp.sum(sm * posf, axis=0, keepdims=True)
    reg_part = jnp.sum(smrow, axis=1, keepdims=True)
    np_part = jnp.sum(posf, axis=1, keepdims=True)

    p = jnp.clip(cls_ref[0], 1e-4, 1.0 - 1e-4)
    fl0 = (-0.25) * p * p * jnp.log(1.0 - p)
    row0 = jnp.sum(fl0, axis=1, keepdims=True)
    c = p.shape[1]
    labc = lab.reshape(blk, 1)
    lane = jax.lax.broadcasted_iota(jnp.int32, (blk, c), 1)
    eql = (lane == labc.astype(jnp.int32)).astype(f32)
    plab = jnp.sum(eql * p, axis=1, keepdims=True)
    plabr = jnp.clip(plab.reshape(1, blk), 1e-4, 1.0 - 1e-4)
    row0r = row0.reshape(1, blk)
    fl1 = (-0.25) * (1.0 - plabr) * (1.0 - plabr) * jnp.log(plabr)
    fl0l = (-0.25) * plabr * plabr * jnp.log(1.0 - plabr)
    cls_part = jnp.sum(incf * row0r + posf * (fl1 - fl0l), axis=1,
                       keepdims=True)

    @pl.when(i == 0)
    def _init():
        cls_out[0] = cls_part
        reg_out[0] = reg_part
        np_out[0] = np_part

    @pl.when(i != 0)
    def _acc():
        cls_out[0] += cls_part
        reg_out[0] += reg_part
        np_out[0] += np_part


def _final(cs_ref, rs_ref, np_ref, co_ref, ro_ref):
    npv = np_ref[...]
    b = npv.shape[0]
    npc = jnp.maximum(npv, 1.0)
    cl = cs_ref[...] / npc
    rl = jnp.where(npv > 0.0, rs_ref[...] / (npc * 4.0), 0.0)
    co_ref[...] = jnp.sum(cl, axis=0, keepdims=True) / float(b)
    ro_ref[...] = jnp.sum(rl, axis=0, keepdims=True) / float(b)


@jax.jit
def kernel(classifications, regressions, anchors, annotations):
    b, n, c = classifications.shape
    m = annotations.shape[1]
    blk = 2000
    nblk = n // blk

    reg_t = jnp.transpose(regressions.reshape(b, nblk, blk, 4),
                          (0, 1, 3, 2))
    ann_t = jnp.transpose(annotations, (0, 2, 1))
    anchor_t = jnp.transpose(anchors[0].reshape(nblk, blk, 4),
                             (0, 2, 1))

    f32 = jnp.float32
    cs, rs, npos = pl.pallas_call(
        _body,
        grid=(b, nblk),
        in_specs=[
            pl.BlockSpec((1, blk, c), lambda bi, ii: (bi, ii, 0)),
            pl.BlockSpec((1, 1, 4, blk), lambda bi, ii: (bi, ii, 0, 0)),
            pl.BlockSpec((1, 4, blk), lambda bi, ii: (ii, 0, 0)),
            pl.BlockSpec((1, 5, m), lambda bi, ii: (bi, 0, 0)),
            pl.BlockSpec((1, m, 5), lambda bi, ii: (bi, 0, 0)),
        ],
        out_specs=[
            pl.BlockSpec((1, 1, 1), lambda bi, ii: (bi, 0, 0)),
            pl.BlockSpec((1, 1, 1), lambda bi, ii: (bi, 0, 0)),
            pl.BlockSpec((1, 1, 1), lambda bi, ii: (bi, 0, 0)),
        ],
        out_shape=[
            jax.ShapeDtypeStruct((b, 1, 1), f32),
            jax.ShapeDtypeStruct((b, 1, 1), f32),
            jax.ShapeDtypeStruct((b, 1, 1), f32),
        ],
        compiler_params=pltpu.CompilerParams(
            dimension_semantics=("parallel", "arbitrary")),
    )(classifications, reg_t, anchor_t, ann_t, annotations)
    cs = cs.reshape(b, 1)
    rs = rs.reshape(b, 1)
    npos = npos.reshape(b, 1)

    co, ro = pl.pallas_call(
        _final,
        out_shape=[
            jax.ShapeDtypeStruct((1, 1), f32),
            jax.ShapeDtypeStruct((1, 1), f32),
        ],
    )(cs, rs, npos)
    return co.reshape(1), ro.reshape(1)
